# Initial kernel scaffold; baseline (speedup 1.0000x reference)
import numpy as np
import jax
import jax.numpy as jnp
from jax import lax
from jax.experimental import pallas as pl
from jax.experimental.pallas import tpu as pltpu
from jax.experimental.pallas import tpu_sc as plsc

NB = 4
S = 2048
D = 1024
N = NB * S
SCALE = 32.0

NC = 2
NS = 16
NW = NC * NS
SPW = S // NW
CH = 16
NQ = SPW // CH


def _build_pe_np():
    pos = np.arange(S, dtype=np.float32)[:, None]
    i = np.arange(0, D, 2, dtype=np.float32)[None, :]
    pe = np.zeros((S, D), dtype=np.float32)
    pe[:, 0::2] = np.sin(pos / np.float32(10000.0) ** (i / np.float32(D)))
    pe[:, 1::2] = np.cos(pos / np.float32(10000.0) ** ((i + 1.0) / np.float32(D)))
    return pe


_PE = _build_pe_np()


def _sc_body(tok_hbm, pe_hbm, table_hbm, out_hbm, idx_v, pe_v, rows_v, gsem):
    wid = lax.axis_index("s") * NC + lax.axis_index("c")
    s0 = pl.multiple_of(wid * SPW, SPW)
    for b in range(NB):
        pltpu.sync_copy(tok_hbm.at[pl.ds(b * S + s0, SPW)],
                        idx_v.at[pl.ds(b * SPW, SPW)])
    pltpu.sync_copy(pe_hbm.at[pl.ds(s0, SPW)], pe_v)
    for b in range(NB):
        for q in range(NQ):
            pltpu.async_copy(
                table_hbm.at[idx_v.at[pl.ds(b * SPW + q * CH, CH)]],
                rows_v, gsem).wait()

            def cbody(c, _):
                sl = pl.ds(lax.mul(c, 16), 16)
                for r in range(CH):
                    rows_v[r, sl] = rows_v[r, sl] * SCALE + pe_v[q * CH + r, sl]
                return 0

            lax.fori_loop(0, D // 16, cbody, 0)
            pltpu.sync_copy(rows_v,
                            out_hbm.at[pl.ds(b * S + s0 + q * CH, CH)])


def _sc_gather(tok_flat, pe, table):
    mesh = plsc.VectorSubcoreMesh(core_axis_name="c", subcore_axis_name="s",
                                  num_cores=NC, num_subcores=NS)
    f = pl.kernel(
        _sc_body,
        out_type=jax.ShapeDtypeStruct((N, D), jnp.float32),
        mesh=mesh,
        scratch_types=[
            pltpu.VMEM((NB * SPW,), jnp.int32),
            pltpu.VMEM((SPW, D), jnp.float32),
            pltpu.VMEM((CH, D), jnp.float32),
            pltpu.SemaphoreType.DMA,
        ],
    )
    return f(tok_flat, pe, table)


_BI = 256


def _mask_body(tok_ref, out_ref):
    i = pl.program_id(1)
    rows = lax.broadcasted_iota(jnp.int32, (_BI, S), 0) + i * _BI
    cols = lax.broadcasted_iota(jnp.int32, (_BI, S), 1)
    out_ref[0] = (cols <= rows) & (tok_ref[0:1, :] != 0)


def _mask(tok):
    return pl.pallas_call(
        _mask_body,
        out_shape=jax.ShapeDtypeStruct((NB, S, S), jnp.bool_),
        grid=(NB, S // _BI),
        in_specs=[pl.BlockSpec((1, S), lambda b, i: (b, 0))],
        out_specs=pl.BlockSpec((1, _BI, S), lambda b, i: (b, i, 0)),
    )(tok)


def kernel(token_ids, emb_table):
    tok = token_ids.astype(jnp.int32)
    pe = jnp.asarray(_PE)
    x = _sc_gather(tok.reshape(-1), pe, emb_table).reshape(NB, S, D)
    return (x, _mask(tok))


# speedup vs baseline: 1.1789x; 1.1789x over previous
import numpy as np
import jax
import jax.numpy as jnp
from jax import lax
from jax.experimental import pallas as pl
from jax.experimental.pallas import tpu as pltpu
from jax.experimental.pallas import tpu_sc as plsc

NB = 4
S = 2048
D = 1024
N = NB * S
SCALE = 32.0

NC = 2
NS = 16
NW = NC * NS
SPW = S // NW
CH = 16
NQ = SPW // CH


def _build_pe_np():
    pos = np.arange(S, dtype=np.float32)[:, None]
    i = np.arange(0, D, 2, dtype=np.float32)[None, :]
    pe = np.zeros((S, D), dtype=np.float32)
    pe[:, 0::2] = np.sin(pos / np.float32(10000.0) ** (i / np.float32(D)))
    pe[:, 1::2] = np.cos(pos / np.float32(10000.0) ** ((i + 1.0) / np.float32(D)))
    return pe


_PE = _build_pe_np()


def _sc_body(tok_hbm, pe_hbm, table_hbm, out_hbm, idx_v, pe_v, rows_v, gsem):
    wid = lax.axis_index("s") * NC + lax.axis_index("c")
    s0 = pl.multiple_of(wid * SPW, SPW)
    for b in range(NB):
        pltpu.sync_copy(tok_hbm.at[pl.ds(b * S + s0, SPW)],
                        idx_v.at[pl.ds(b * SPW, SPW)])
    pltpu.sync_copy(pe_hbm.at[pl.ds(s0, SPW)], pe_v)
    for b in range(NB):
        for q in range(NQ):
            pltpu.async_copy(
                table_hbm.at[idx_v.at[pl.ds(b * SPW + q * CH, CH)]],
                rows_v, gsem).wait()

            def cbody(c, _):
                sl = pl.ds(lax.mul(c, 16), 16)
                for r in range(CH):
                    rows_v[r, sl] = rows_v[r, sl] * SCALE + pe_v[q * CH + r, sl]
                return 0

            lax.fori_loop(0, D // 16, cbody, 0)
            pltpu.sync_copy(rows_v,
                            out_hbm.at[pl.ds(b * S + s0 + q * CH, CH)])


def _sc_gather(tok_flat, pe, table):
    mesh = plsc.VectorSubcoreMesh(core_axis_name="c", subcore_axis_name="s",
                                  num_cores=NC, num_subcores=NS)
    f = pl.kernel(
        _sc_body,
        out_type=jax.ShapeDtypeStruct((N, D), jnp.float32),
        mesh=mesh,
        scratch_types=[
            pltpu.VMEM((NB * SPW,), jnp.int32),
            pltpu.VMEM((SPW, D), jnp.float32),
            pltpu.VMEM((CH, D), jnp.float32),
            pltpu.SemaphoreType.DMA,
        ],
    )
    return f(tok_flat, pe, table)


_BI = 256


def _mask_body(tok_ref, out_ref):
    i = pl.program_id(1)
    rows = lax.broadcasted_iota(jnp.int32, (_BI, S), 0) + i * _BI
    cols = lax.broadcasted_iota(jnp.int32, (_BI, S), 1)
    out_ref[0] = (cols <= rows) & (tok_ref[0, 0:1, :] != 0)


def _mask(tok):
    return pl.pallas_call(
        _mask_body,
        out_shape=jax.ShapeDtypeStruct((NB, S, S), jnp.bool_),
        grid=(NB, S // _BI),
        in_specs=[pl.BlockSpec((1, 1, S), lambda b, i: (b, 0, 0))],
        out_specs=pl.BlockSpec((1, _BI, S), lambda b, i: (b, i, 0)),
    )(tok.reshape(NB, 1, S))


def kernel(token_ids, emb_table):
    tok = token_ids.astype(jnp.int32)
    pe = jnp.asarray(_PE)
    x = _sc_gather(tok.reshape(-1), pe, emb_table).reshape(NB, S, D)
    return (x, _mask(tok))
